# baseline (speedup 1.0000x reference)
import functools

import jax
import jax.numpy as jnp
from jax import lax
from jax.experimental import pallas as pl
from jax.experimental.pallas import tpu as pltpu
from jax.experimental.pallas import tpu_sc as plsc

_NC = 2
_NS = 16
_L = 16


@functools.lru_cache(maxsize=None)
def _make_kernel(N, V, D, C, NBUF, AH):
    NW = _NC * _NS
    per_w = N // NW
    nchunk = per_w // C
    assert nchunk % NBUF == 0 and AH < NBUF
    mesh = plsc.VectorSubcoreMesh(core_axis_name="c", subcore_axis_name="s")

    @functools.partial(
        pl.kernel,
        mesh=mesh,
        out_type=jax.ShapeDtypeStruct((N, D), jnp.float32),
        scratch_types=[
            pltpu.VMEM((per_w,), jnp.int32),
            pltpu.VMEM((per_w + _L,), jnp.int32),
            pltpu.VMEM((NBUF, C, D), jnp.float32),
        ]
        + [pltpu.SemaphoreType.DMA] * (2 * NBUF),
    )
    def k(ids_hbm, mask_hbm, table_hbm, out_hbm, idx_v, msk_v, rows_v, *sems):
        gsem, ssem = sems[:NBUF], sems[NBUF:]
        wid = lax.axis_index("s") * _NC + lax.axis_index("c")
        base = wid * per_w
        pltpu.sync_copy(ids_hbm.at[pl.ds(base, per_w)], idx_v)
        pltpu.sync_copy(mask_hbm.at[pl.ds(base, per_w)], msk_v.at[pl.ds(0, per_w)])

        def mul_body(i, _):
            s = pl.ds(i * _L, _L)
            idx_v[s] = idx_v[s] * msk_v[s]
            return 0

        lax.fori_loop(0, per_w // _L, mul_body, 0, unroll=4)

        def gather(c, b):
            return pltpu.make_async_copy(
                table_hbm.at[idx_v.at[pl.ds(c * C, C)]], rows_v.at[b], gsem[b]
            )

        def store(c, b):
            return pltpu.make_async_copy(
                rows_v.at[b], out_hbm.at[pl.ds(base + c * C, C)], ssem[b]
            )

        def zero_masked(c, b):
            mvec = msk_v[pl.ds(c * C, _L)]
            for i in range(C):

                @pl.when(mvec[i] == 0)
                def _zero(i=i):
                    def col_body(j, _):
                        rows_v[b, i, pl.ds(j * _L, _L)] = jnp.zeros(
                            (_L,), jnp.float32
                        )
                        return 0

                    lax.fori_loop(0, D // _L, col_body, 0, unroll=8)

        for c in range(AH):
            gather(c, c).start()

        def rot_body(r, _):
            for b in range(NBUF):
                c = r * NBUF + b
                ba_m = (b - AH) % NBUF
                ba_p = (b + AH) % NBUF

                @pl.when(c - AH >= 0)
                def _retire():
                    store(c - AH, ba_m).wait()

                @pl.when(c + AH < nchunk)
                def _prefetch():
                    gather(c + AH, ba_p).start()

                gather(c, b).wait()
                zero_masked(c, b)
                store(c, b).start()
            return 0

        lax.fori_loop(0, nchunk // NBUF, rot_body, 0)
        for c in range(nchunk - AH, nchunk):
            store(c, c % NBUF).wait()

    return k


@jax.jit
def kernel(input_ids, input_mask, embedding_table):
    B, S = input_ids.shape
    V, D = embedding_table.shape
    N = B * S
    ids = input_ids.reshape(N)
    msk = input_mask.reshape(N)
    out = _make_kernel(N, V, D, 8, 4, 2)(ids, msk, embedding_table)
    return out.reshape(B, S, D)


# speedup vs baseline: 1.4975x; 1.4975x over previous
import functools

import jax
import jax.numpy as jnp
from jax import lax
from jax.experimental import pallas as pl
from jax.experimental.pallas import tpu as pltpu
from jax.experimental.pallas import tpu_sc as plsc

_NC = 2
_NS = 16
_L = 16


@functools.lru_cache(maxsize=None)
def _make_kernel(N, V, D, C, NBUF, AH):
    NW = _NC * _NS
    per_w = N // NW
    nchunk = per_w // C
    assert nchunk % NBUF == 0 and AH < NBUF
    mesh = plsc.VectorSubcoreMesh(core_axis_name="c", subcore_axis_name="s")

    @functools.partial(
        pl.kernel,
        mesh=mesh,
        out_type=jax.ShapeDtypeStruct((N, D), jnp.float32),
        scratch_types=[
            pltpu.VMEM((per_w + _L,), jnp.int32),
            pltpu.VMEM((per_w + _L,), jnp.int32),
            pltpu.VMEM_SHARED((_NS, NBUF, C, D), jnp.float32),
        ]
        + [pltpu.SemaphoreType.DMA] * (2 * NBUF),
    )
    def k(ids_hbm, mask_hbm, table_hbm, zero_hbm, out_hbm, idx_v, msk_v,
          rows_sh, *sems):
        gsem, ssem = sems[:NBUF], sems[NBUF:]
        s_id = lax.axis_index("s")
        wid = s_id * _NC + lax.axis_index("c")
        base = wid * per_w
        pltpu.sync_copy(ids_hbm.at[pl.ds(base, per_w)], idx_v.at[pl.ds(0, per_w)])
        pltpu.sync_copy(mask_hbm.at[pl.ds(base, per_w)], msk_v.at[pl.ds(0, per_w)])

        def mul_body(i, _):
            s = pl.ds(i * _L, _L)
            idx_v[s] = idx_v[s] * msk_v[s]
            return 0

        lax.fori_loop(0, per_w // _L, mul_body, 0, unroll=4)

        def gather_rows(c, b):
            gvec = idx_v[pl.ds(c * C, _L)]
            mvec = msk_v[pl.ds(c * C, _L)]
            for i in range(C):
                dst = rows_sh.at[s_id, b, i]

                @pl.when(mvec[i] == 0)
                def _z(dst=dst):
                    pltpu.make_async_copy(zero_hbm.at[0], dst, gsem[b]).start()

                @pl.when(mvec[i] != 0)
                def _g(dst=dst, i=i):
                    pltpu.make_async_copy(table_hbm.at[gvec[i]], dst, gsem[b]).start()

        def gather_drain(c, b):
            for i in range(C):
                pltpu.make_async_copy(
                    zero_hbm.at[0], rows_sh.at[s_id, b, i], gsem[b]
                ).wait()

        def store(c, b):
            return pltpu.make_async_copy(
                rows_sh.at[s_id, b], out_hbm.at[pl.ds(base + c * C, C)], ssem[b]
            )

        for c in range(AH):
            gather_rows(c, c)

        def rot_body(r, _):
            for b in range(NBUF):
                c = r * NBUF + b
                ba_m = (b - AH) % NBUF
                ba_p = (b + AH) % NBUF

                @pl.when(c - AH >= 0)
                def _retire():
                    store(c - AH, ba_m).wait()

                @pl.when(c + AH < nchunk)
                def _prefetch():
                    gather_rows(c + AH, ba_p)

                gather_drain(c, b)
                store(c, b).start()
            return 0

        lax.fori_loop(0, nchunk // NBUF, rot_body, 0)
        for c in range(nchunk - AH, nchunk):
            store(c, c % NBUF).wait()

    return k


@jax.jit
def kernel(input_ids, input_mask, embedding_table):
    B, S = input_ids.shape
    V, D = embedding_table.shape
    N = B * S
    ids = input_ids.reshape(N)
    msk = input_mask.reshape(N)
    zrow = jnp.zeros((1, D), jnp.float32)
    out = _make_kernel(N, V, D, 8, 4, 2)(ids, msk, embedding_table, zrow)
    return out.reshape(B, S, D)


# speedup vs baseline: 7.1953x; 4.8050x over previous
import functools

import jax
import jax.numpy as jnp
from jax import lax
from jax.experimental import pallas as pl
from jax.experimental.pallas import tpu as pltpu
from jax.experimental.pallas import tpu_sc as plsc

_NC = 2
_NS = 16
_L = 16


@functools.lru_cache(maxsize=None)
def _make_kernel(N, V, D, C, NBUF, AH):
    NW = _NC * _NS
    per_w = N // NW
    nchunk = per_w // C
    assert nchunk % NBUF == 0 and AH < NBUF
    mesh = plsc.VectorSubcoreMesh(core_axis_name="c", subcore_axis_name="s")

    @functools.partial(
        pl.kernel,
        mesh=mesh,
        out_type=jax.ShapeDtypeStruct((N, D), jnp.float32),
        scratch_types=[
            pltpu.VMEM((per_w + _L,), jnp.int32),
            pltpu.VMEM((per_w + _L,), jnp.int32),
            pltpu.VMEM((NBUF, C, D), jnp.float32),
        ]
        + [pltpu.SemaphoreType.DMA] * (2 * NBUF),
    )
    def k(ids_hbm, mask_hbm, table_hbm, out_hbm, idx_v, msk_v, rows_v, *sems):
        gsem, ssem = sems[:NBUF], sems[NBUF:]
        wid = lax.axis_index("s") * _NC + lax.axis_index("c")
        base = wid * per_w
        pltpu.sync_copy(ids_hbm.at[pl.ds(base, per_w)], idx_v.at[pl.ds(0, per_w)])
        pltpu.sync_copy(mask_hbm.at[pl.ds(base, per_w)], msk_v.at[pl.ds(0, per_w)])

        def mul_body(i, _):
            s = pl.ds(i * _L, _L)
            idx_v[s] = idx_v[s] * msk_v[s]
            return 0

        lax.fori_loop(0, per_w // _L, mul_body, 0, unroll=4)

        def gather_rows(c, b):
            gvec = idx_v[pl.ds(c * C, _L)]
            mvec = msk_v[pl.ds(c * C, _L)]
            for i in range(C):

                @pl.when(mvec[i] == 0)
                def _z(i=i):
                    def col_body(j, _):
                        rows_v[b, i, pl.ds(j * _L, _L)] = jnp.zeros(
                            (_L,), jnp.float32
                        )
                        return 0

                    lax.fori_loop(0, D // _L, col_body, 0, unroll=8)

                @pl.when(mvec[i] != 0)
                def _g(i=i):
                    pltpu.make_async_copy(
                        table_hbm.at[gvec[i]], rows_v.at[b, i], gsem[b]
                    ).start()

        def gather_drain(c, b):
            mvec = msk_v[pl.ds(c * C, _L)]
            for i in range(C):

                @pl.when(mvec[i] != 0)
                def _g(i=i):
                    pltpu.make_async_copy(
                        table_hbm.at[pl.ds(0, 1)], rows_v.at[b, pl.ds(i, 1)],
                        gsem[b]
                    ).wait()

        def store(c, b):
            return pltpu.make_async_copy(
                rows_v.at[b], out_hbm.at[pl.ds(base + c * C, C)], ssem[b]
            )

        for c in range(AH):
            gather_rows(c, c)

        def rot_body(r, _):
            for b in range(NBUF):
                c = r * NBUF + b
                ba_m = (b - AH) % NBUF
                ba_p = (b + AH) % NBUF

                @pl.when(c - AH >= 0)
                def _retire():
                    store(c - AH, ba_m).wait()

                @pl.when(c + AH < nchunk)
                def _prefetch():
                    gather_rows(c + AH, ba_p)

                gather_drain(c, b)
                store(c, b).start()
            return 0

        lax.fori_loop(0, nchunk // NBUF, rot_body, 0)
        for c in range(nchunk - AH, nchunk):
            store(c, c % NBUF).wait()

    return k


@jax.jit
def kernel(input_ids, input_mask, embedding_table):
    B, S = input_ids.shape
    V, D = embedding_table.shape
    N = B * S
    ids = input_ids.reshape(N)
    msk = input_mask.reshape(N)
    out = _make_kernel(N, V, D, 8, 4, 2)(ids, msk, embedding_table)
    return out.reshape(B, S, D)


# speedup vs baseline: 7.2263x; 1.0043x over previous
import functools

import jax
import jax.numpy as jnp
from jax import lax
from jax.experimental import pallas as pl
from jax.experimental.pallas import tpu as pltpu
from jax.experimental.pallas import tpu_sc as plsc

_NC = 2
_NS = 16
_L = 16


@functools.lru_cache(maxsize=None)
def _make_kernel(N, V, D, C, NBUF, AH):
    NW = _NC * _NS
    per_w = N // NW
    nchunk = per_w // C
    assert nchunk % NBUF == 0 and AH < NBUF
    mesh = plsc.VectorSubcoreMesh(core_axis_name="c", subcore_axis_name="s")

    @functools.partial(
        pl.kernel,
        mesh=mesh,
        out_type=jax.ShapeDtypeStruct((N, D), jnp.float32),
        scratch_types=[
            pltpu.VMEM((per_w + _L,), jnp.int32),
            pltpu.VMEM((per_w + _L,), jnp.int32),
            pltpu.VMEM((NBUF, C, D), jnp.float32),
        ]
        + [pltpu.SemaphoreType.DMA] * (2 * NBUF),
    )
    def k(ids_hbm, mask_hbm, table_hbm, out_hbm, idx_v, msk_v, rows_v, *sems):
        gsem, ssem = sems[:NBUF], sems[NBUF:]
        wid = lax.axis_index("s") * _NC + lax.axis_index("c")
        base = wid * per_w
        pltpu.sync_copy(ids_hbm.at[pl.ds(base, per_w)], idx_v.at[pl.ds(0, per_w)])
        pltpu.sync_copy(mask_hbm.at[pl.ds(base, per_w)], msk_v.at[pl.ds(0, per_w)])

        def gather_rows(c, b):
            gvec = idx_v[pl.ds(c * C, _L)]
            mvec = msk_v[pl.ds(c * C, _L)]
            for i in range(C):

                @pl.when(mvec[i] == 0)
                def _z(i=i):
                    def col_body(j, _):
                        rows_v[b, i, pl.ds(j * _L, _L)] = jnp.zeros(
                            (_L,), jnp.float32
                        )
                        return 0

                    lax.fori_loop(0, D // _L, col_body, 0, unroll=8)

                @pl.when(mvec[i] != 0)
                def _g(i=i):
                    pltpu.make_async_copy(
                        table_hbm.at[gvec[i]], rows_v.at[b, i], gsem[b]
                    ).start()

        def gather_drain(c, b):
            mvec = msk_v[pl.ds(c * C, _L)]
            for i in range(C):

                @pl.when(mvec[i] != 0)
                def _g(i=i):
                    pltpu.make_async_copy(
                        table_hbm.at[pl.ds(0, 1)], rows_v.at[b, pl.ds(i, 1)],
                        gsem[b]
                    ).wait()

        def store(c, b):
            return pltpu.make_async_copy(
                rows_v.at[b], out_hbm.at[pl.ds(base + c * C, C)], ssem[b]
            )

        for c in range(AH):
            gather_rows(c, c)

        def rot_body(r, _):
            for b in range(NBUF):
                c = r * NBUF + b
                ba_m = (b - AH) % NBUF
                ba_p = (b + AH) % NBUF

                @pl.when(c - AH >= 0)
                def _retire():
                    store(c - AH, ba_m).wait()

                @pl.when(c + AH < nchunk)
                def _prefetch():
                    gather_rows(c + AH, ba_p)

                gather_drain(c, b)
                store(c, b).start()
            return 0

        lax.fori_loop(0, nchunk // NBUF, rot_body, 0)
        for c in range(nchunk - AH, nchunk):
            store(c, c % NBUF).wait()

    return k


@jax.jit
def kernel(input_ids, input_mask, embedding_table):
    B, S = input_ids.shape
    V, D = embedding_table.shape
    N = B * S
    ids = input_ids.reshape(N)
    msk = input_mask.reshape(N)
    out = _make_kernel(N, V, D, 8, 4, 2)(ids, msk, embedding_table)
    return out.reshape(B, S, D)
